# Initial kernel scaffold; baseline (speedup 1.0000x reference)
import jax
import jax.numpy as jnp
from jax.experimental import pallas as pl


def kernel(x, linear_w, v_w, b):
    raise NotImplementedError("write your pallas kernel here")


# speedup vs baseline: 13.8932x; 13.8932x over previous
import functools

import jax
import jax.numpy as jnp
from jax import lax
from jax.experimental import pallas as pl
from jax.experimental.pallas import tpu as pltpu
from jax.experimental.pallas import tpu_sc as plsc

F = 26
K = 16
NC = 2
NS = 16
NW = NC * NS
CH = 4
NBUF = 2


def _ffm_kernel(B, EPW, NCHUNK):
    mesh = plsc.VectorSubcoreMesh(core_axis_name="c", subcore_axis_name="s",
                                  num_cores=NC, num_subcores=NS)

    @functools.partial(
        pl.kernel,
        mesh=mesh,
        compiler_params=pltpu.CompilerParams(needs_layout_passes=False,
                                             use_tc_tiling_on_sc=False),
        out_type=jax.ShapeDtypeStruct((B,), jnp.float32),
        scratch_types=[
            pltpu.VMEM((EPW, F), jnp.int32),
            pltpu.VMEM((F, EPW), jnp.int32),
            pltpu.VMEM((NBUF, CH, F, F * K), jnp.float32),
            pltpu.VMEM((F, EPW), jnp.float32),
            pltpu.VMEM((EPW,), jnp.float32),
            pltpu.VMEM((K,), jnp.float32),
            pltpu.SemaphoreType.DMA,
            pltpu.SemaphoreType.DMA,
            pltpu.SemaphoreType.DMA,
        ],
    )
    def body(x_hbm, xt_hbm, vtab_hbm, ltab_hbm, b_hbm, out_hbm,
             xw, xt, big, lvals, res, bv, sem0, sem1, lsem):
        sems = (sem0, sem1)
        wid = lax.axis_index("s") * NC + lax.axis_index("c")
        base = wid * EPW
        NV = EPW // K

        pltpu.sync_copy(x_hbm.at[pl.ds(base, EPW)], xw)
        pltpu.sync_copy(xt_hbm.at[:, pl.ds(base, EPW)], xt)
        pltpu.sync_copy(b_hbm, bv.at[pl.ds(0, 1)])
        b0 = bv[pl.ds(0, K)][0]

        for i in range(F):
            pltpu.async_copy(ltab_hbm.at[xt.at[i]], lvals.at[i], lsem)

        def fire(chunk, par):
            for e in range(CH):
                idx = xw.at[chunk * CH + e]
                pltpu.async_copy(vtab_hbm.at[idx], big.at[par, e], sems[par])

        def drain(chunk, par):
            for e in range(CH):
                idx = xw.at[chunk * CH + e]
                pltpu.make_async_copy(vtab_hbm.at[idx], big.at[par, e],
                                      sems[par]).wait()

        lane0 = lax.iota(jnp.int32, 16) == 0

        def compute(chunk, par):
            def elem(e, carry):
                accs = [jnp.zeros((K,), jnp.float32) for _ in range(4)]
                n = 0
                for i in range(F):
                    for j in range(i + 1, F):
                        a = big[par, e, i, pl.ds(j * K, K)]
                        c = big[par, e, j, pl.ds(i * K, K)]
                        accs[n % 4] = accs[n % 4] + a * c
                        n += 1
                acc = (accs[0] + accs[1]) + (accs[2] + accs[3])
                tot = jnp.sum(acc)
                slot = jnp.full((16,), chunk * CH + e, jnp.int32)
                plsc.store_scatter(res, [slot],
                                   jnp.full((16,), tot, jnp.float32),
                                   mask=lane0)
                return carry

            lax.fori_loop(0, CH, elem, 0)

        for par in range(NBUF):
            fire(par, par)

        def outer(g, carry):
            for par in range(NBUF):
                chunk = NBUF * g + par
                drain(chunk, par)
                compute(chunk, par)
                nxt = chunk + NBUF

                @pl.when(nxt < NCHUNK)
                def _():
                    fire(nxt, par)
            return carry

        lax.fori_loop(0, NCHUNK // NBUF, outer, 0)

        for i in range(F):
            pltpu.make_async_copy(ltab_hbm.at[xt.at[i]], lvals.at[i],
                                  lsem).wait()

        for v in range(NV):
            lacc = jnp.full((K,), b0, jnp.float32)
            for i in range(F):
                lacc = lacc + lvals[i, pl.ds(v * K, K)]
            res[pl.ds(v * K, K)] = res[pl.ds(v * K, K)] + lacc

        pltpu.sync_copy(res, out_hbm.at[pl.ds(base, EPW)])

    return body


@jax.jit
def kernel(x, linear_w, v_w, b):
    B, f = x.shape
    n, _, k = v_w.shape
    vtab = v_w.reshape(n, f * k)
    ltab = linear_w.reshape(n)
    EPW = B // NW
    NCHUNK = EPW // CH
    out = _ffm_kernel(B, EPW, NCHUNK)(x, x.T, vtab, ltab, b)
    return out.reshape(B, 1)
